# Initial kernel scaffold; baseline (speedup 1.0000x reference)
import jax
import jax.numpy as jnp
from jax.experimental import pallas as pl


def kernel(test, question, tag, correct, mask, interaction, duration, emb_inter, emb_test, emb_q, emb_tag, Wc, bc, Wih0, Whh0, bih0, bhh0, Wih1, Whh1, bih1, bhh1, Wf, bf):
    raise NotImplementedError("write your pallas kernel here")


# speedup vs baseline: 1.4708x; 1.4708x over previous
import functools

import jax
import jax.numpy as jnp
from jax import lax
from jax.experimental import pallas as pl
from jax.experimental.pallas import tpu as pltpu
from jax.experimental.pallas import tpu_sc as plsc

B, T, H = 1024, 50, 96
E = 32
G4 = 4 * H
FE = 4 * E

NC, NS = 2, 16
NW = NC * NS
BT = B * T
RPW = BT // NW
CH = 80
NCH = RPW // CH

BB = 256
NB = B // BB


def _sc_gather_body(idx_hbm, t_inter, t_test, t_q, t_tag, out_hbm,
                    idx_v, rows_v, sem):
    wid = lax.axis_index("s") * NC + lax.axis_index("c")
    base = wid * RPW
    tables = (t_inter, t_test, t_q, t_tag)
    for j, tab in enumerate(tables):
        pltpu.sync_copy(idx_hbm.at[j, wid], idx_v)
        copies = []
        for c in range(NCH):
            copies.append(
                pltpu.async_copy(tab.at[idx_v.at[c]],
                                 rows_v.at[pl.ds(c * CH, CH)], sem))
        for cp in copies:
            cp.wait()
        pltpu.sync_copy(rows_v,
                        out_hbm.at[pl.ds(base, RPW), pl.ds(j * E, E)])


@functools.partial(jax.jit, static_argnums=())
def _sc_gather(idx, emb_inter, emb_test, emb_q, emb_tag):
    mesh = plsc.VectorSubcoreMesh(core_axis_name="c", subcore_axis_name="s")
    return pl.kernel(
        _sc_gather_body,
        out_type=jax.ShapeDtypeStruct((BT, FE), jnp.float32),
        mesh=mesh,
        compiler_params=pltpu.CompilerParams(use_tc_tiling_on_sc=False),
        scratch_types=[
            pltpu.VMEM((NCH, CH), jnp.int32),
            pltpu.VMEM((RPW, E), jnp.float32),
            pltpu.SemaphoreType.DMA,
        ],
    )(idx, emb_inter, emb_test, emb_q, emb_tag)


def _tc_body(e_ref, Wc_ref, bc_ref, Wih0_ref, Whh0_ref, b0_ref,
             Wih1_ref, Whh1_ref, b1_ref, Wf_ref, bf_ref,
             out_ref, Xg_ref, hseq_ref):
    cdims = (((1,), (1,)), ((), ()))

    ecat = e_ref[...].reshape(T * BB, FE)
    X = lax.dot_general(ecat, Wc_ref[...], cdims,
                        preferred_element_type=jnp.float32) + bc_ref[...]
    Xg_ref[...] = (lax.dot_general(X, Wih0_ref[...], cdims,
                                   preferred_element_type=jnp.float32)
                   + b0_ref[...]).reshape(T, BB, G4)

    def recur(Whh_ref):
        def step(t, carry):
            h, c = carry
            g = Xg_ref[t] + lax.dot_general(
                h, Whh_ref[...], cdims, preferred_element_type=jnp.float32)
            i = jax.nn.sigmoid(g[:, 0:H])
            f = jax.nn.sigmoid(g[:, H:2 * H])
            gg = jnp.tanh(g[:, 2 * H:3 * H])
            o = jax.nn.sigmoid(g[:, 3 * H:4 * H])
            c = f * c + i * gg
            h = o * jnp.tanh(c)
            hseq_ref[t] = h
            return (h, c)
        z = jnp.zeros((BB, H), jnp.float32)
        lax.fori_loop(0, T, step, (z, z))

    recur(Whh0_ref)
    Xg_ref[...] = (lax.dot_general(hseq_ref[...].reshape(T * BB, H),
                                   Wih1_ref[...], cdims,
                                   preferred_element_type=jnp.float32)
                   + b1_ref[...]).reshape(T, BB, G4)
    recur(Whh1_ref)
    out_ref[...] = (jnp.sum(hseq_ref[...] * Wf_ref[...][0], axis=-1)
                    + bf_ref[0, 0])


def _tc_lstm(e_tm, Wc, bc, Wih0, Whh0, b0, Wih1, Whh1, b1, Wf, bf,
             interpret=False):
    full = lambda shape: pl.BlockSpec(shape, lambda i: (0,) * len(shape))
    return pl.pallas_call(
        _tc_body,
        grid=(NB,),
        in_specs=[
            pl.BlockSpec((T, BB, FE), lambda i: (0, i, 0)),
            full((H, FE)), full((1, H)),
            full((G4, H)), full((G4, H)), full((1, G4)),
            full((G4, H)), full((G4, H)), full((1, G4)),
            full((1, H)), full((1, 1)),
        ],
        out_specs=pl.BlockSpec((T, BB), lambda i: (0, i)),
        out_shape=jax.ShapeDtypeStruct((T, B), jnp.float32),
        scratch_shapes=[
            pltpu.VMEM((T, BB, G4), jnp.float32),
            pltpu.VMEM((T, BB, H), jnp.float32),
        ],
        interpret=interpret,
    )(e_tm, Wc, bc, Wih0, Whh0, b0, Wih1, Whh1, b1, Wf, bf)


def kernel(test, question, tag, correct, mask, interaction, duration,
           emb_inter, emb_test, emb_q, emb_tag, Wc, bc,
           Wih0, Whh0, bih0, bhh0, Wih1, Whh1, bih1, bhh1, Wf, bf):
    idx = jnp.stack([
        interaction.T.reshape(-1), test.T.reshape(-1),
        question.T.reshape(-1), tag.T.reshape(-1),
    ]).reshape(4, NW, NCH, CH)
    e = _sc_gather(idx, emb_inter, emb_test, emb_q, emb_tag)
    e_tm = e.reshape(T, B, FE)
    out_tm = _tc_lstm(
        e_tm, Wc, bc.reshape(1, H),
        Wih0, Whh0, (bih0 + bhh0).reshape(1, G4),
        Wih1, Whh1, (bih1 + bhh1).reshape(1, G4),
        Wf, bf.reshape(1, 1))
    return out_tm.T


# speedup vs baseline: 1.4837x; 1.0088x over previous
import functools

import jax
import jax.numpy as jnp
from jax import lax
from jax.experimental import pallas as pl
from jax.experimental.pallas import tpu as pltpu
from jax.experimental.pallas import tpu_sc as plsc

B, T, H = 1024, 50, 96
E = 32
G4 = 4 * H
FE = 4 * E

NC, NS = 2, 16
NW = NC * NS
BT = B * T
RPW = BT // NW
CH = 80
NCH = RPW // CH

BB = 256
NB = B // BB


def _sc_gather_body(idx_hbm, t_inter, t_test, t_q, t_tag, out_hbm,
                    idx_v, rows_v, gsem, osem):
    wid = lax.axis_index("s") * NC + lax.axis_index("c")
    base = wid * RPW
    tables = (t_inter, t_test, t_q, t_tag)
    pltpu.sync_copy(idx_hbm.at[wid], idx_v)
    out_copies = [None, None]
    gather_waves = []
    for j, tab in enumerate(tables):
        s = j % 2
        if out_copies[s] is not None:
            out_copies[s].wait()
        copies = []
        for c in range(NCH):
            copies.append(
                pltpu.async_copy(tab.at[idx_v.at[j, c]],
                                 rows_v.at[s, pl.ds(c * CH, CH)], gsem))
        gather_waves.append(copies)
        if j >= 1:
            for cp in gather_waves[j - 1]:
                cp.wait()
            out_copies[(j - 1) % 2] = pltpu.async_copy(
                rows_v.at[(j - 1) % 2],
                out_hbm.at[pl.ds(base, RPW), pl.ds((j - 1) * E, E)], osem)
    for cp in gather_waves[3]:
        cp.wait()
    out_copies[1] = pltpu.async_copy(
        rows_v.at[1], out_hbm.at[pl.ds(base, RPW), pl.ds(3 * E, E)], osem)
    out_copies[0].wait()
    out_copies[1].wait()


@functools.partial(jax.jit, static_argnums=())
def _sc_gather(idx, emb_inter, emb_test, emb_q, emb_tag):
    mesh = plsc.VectorSubcoreMesh(core_axis_name="c", subcore_axis_name="s")
    return pl.kernel(
        _sc_gather_body,
        out_type=jax.ShapeDtypeStruct((BT, FE), jnp.float32),
        mesh=mesh,
        compiler_params=pltpu.CompilerParams(use_tc_tiling_on_sc=False),
        scratch_types=[
            pltpu.VMEM((4, NCH, CH), jnp.int32),
            pltpu.VMEM((2, RPW, E), jnp.float32),
            pltpu.SemaphoreType.DMA,
            pltpu.SemaphoreType.DMA,
        ],
    )(idx, emb_inter, emb_test, emb_q, emb_tag)


def _tc_body(e_ref, Wc_ref, bc_ref, Wih0_ref, Whh0_ref, b0_ref,
             Wih1_ref, Whh1_ref, b1_ref, Wf_ref, bf_ref,
             out_ref, Xg_ref, hseq_ref):
    cdims = (((1,), (1,)), ((), ()))

    ecat = e_ref[...].reshape(T * BB, FE)
    X = lax.dot_general(ecat, Wc_ref[...], cdims,
                        preferred_element_type=jnp.float32) + bc_ref[...]
    Xg_ref[...] = (lax.dot_general(X, Wih0_ref[...], cdims,
                                   preferred_element_type=jnp.float32)
                   + b0_ref[...]).reshape(T, BB, G4)

    def recur(Whh_ref):
        def step(t, carry):
            h, c = carry
            g = Xg_ref[t] + lax.dot_general(
                h, Whh_ref[...], cdims, preferred_element_type=jnp.float32)
            i = jax.nn.sigmoid(g[:, 0:H])
            f = jax.nn.sigmoid(g[:, H:2 * H])
            gg = jnp.tanh(g[:, 2 * H:3 * H])
            o = jax.nn.sigmoid(g[:, 3 * H:4 * H])
            c = f * c + i * gg
            h = o * jnp.tanh(c)
            hseq_ref[t] = h
            return (h, c)
        z = jnp.zeros((BB, H), jnp.float32)
        lax.fori_loop(0, T, step, (z, z))

    recur(Whh0_ref)
    Xg_ref[...] = (lax.dot_general(hseq_ref[...].reshape(T * BB, H),
                                   Wih1_ref[...], cdims,
                                   preferred_element_type=jnp.float32)
                   + b1_ref[...]).reshape(T, BB, G4)
    recur(Whh1_ref)
    out_ref[...] = (jnp.sum(hseq_ref[...] * Wf_ref[...][0], axis=-1)
                    + bf_ref[0, 0])


def _tc_lstm(e_tm, Wc, bc, Wih0, Whh0, b0, Wih1, Whh1, b1, Wf, bf,
             interpret=False):
    full = lambda shape: pl.BlockSpec(shape, lambda i: (0,) * len(shape))
    return pl.pallas_call(
        _tc_body,
        grid=(NB,),
        in_specs=[
            pl.BlockSpec((T, BB, FE), lambda i: (0, i, 0)),
            full((H, FE)), full((1, H)),
            full((G4, H)), full((G4, H)), full((1, G4)),
            full((G4, H)), full((G4, H)), full((1, G4)),
            full((1, H)), full((1, 1)),
        ],
        out_specs=pl.BlockSpec((T, BB), lambda i: (0, i)),
        out_shape=jax.ShapeDtypeStruct((T, B), jnp.float32),
        scratch_shapes=[
            pltpu.VMEM((T, BB, G4), jnp.float32),
            pltpu.VMEM((T, BB, H), jnp.float32),
        ],
        interpret=interpret,
    )(e_tm, Wc, bc, Wih0, Whh0, b0, Wih1, Whh1, b1, Wf, bf)


def kernel(test, question, tag, correct, mask, interaction, duration,
           emb_inter, emb_test, emb_q, emb_tag, Wc, bc,
           Wih0, Whh0, bih0, bhh0, Wih1, Whh1, bih1, bhh1, Wf, bf):
    idx = jnp.stack([
        interaction.T.reshape(-1), test.T.reshape(-1),
        question.T.reshape(-1), tag.T.reshape(-1),
    ]).reshape(4, NW, NCH, CH).transpose(1, 0, 2, 3)
    e = _sc_gather(idx, emb_inter, emb_test, emb_q, emb_tag)
    e_tm = e.reshape(T, B, FE)
    out_tm = _tc_lstm(
        e_tm, Wc, bc.reshape(1, H),
        Wih0, Whh0, (bih0 + bhh0).reshape(1, G4),
        Wih1, Whh1, (bih1 + bhh1).reshape(1, G4),
        Wf, bf.reshape(1, 1))
    return out_tm.T
